# baseline (speedup 1.0000x reference)
import functools

import jax
import jax.numpy as jnp
from jax import lax
from jax.experimental import pallas as pl
from jax.experimental.pallas import tpu as pltpu
from jax.experimental.pallas import tpu_sc as plsc

NC = 2
NS = 16
NW = NC * NS
CH = 80


K0 = 100
K1 = 152


def _sc_aggregate(support, src, dst, ew):
    N, D = support.shape
    E = src.shape[0]
    e_pad = NS * (K0 + K1) * CH
    pad = e_pad - E
    assert pad >= 0 and K0 % 2 == 0 and K1 % 2 == 0
    if pad:
        src = jnp.pad(src, (0, pad))
        dst = jnp.pad(dst, (0, pad))
        ew = jnp.pad(ew, (0, pad))
    wmax = max(K0, K1) * CH
    assert N % NS == 0
    rows_per_tile = N // NS
    ZR = 125 if CH >= 125 else 25
    assert rows_per_tile % ZR == 0 and ZR <= CH
    nseg = D // 16

    mesh = plsc.VectorSubcoreMesh(core_axis_name="c", subcore_axis_name="s")

    @functools.partial(
        pl.kernel,
        mesh=mesh,
        out_type=jax.ShapeDtypeStruct((NC, N, D), jnp.float32),
        scratch_types=[
            pltpu.VMEM((CH,), jnp.int32),
            pltpu.VMEM((CH,), jnp.int32),
            pltpu.VMEM((CH,), jnp.int32),
            pltpu.VMEM((CH,), jnp.int32),
            pltpu.VMEM((wmax,), jnp.float32),
            pltpu.VMEM((CH, D), jnp.float32),
            pltpu.VMEM((CH, D), jnp.float32),
            pltpu.VMEM_SHARED((N, D), jnp.float32),
            pltpu.SemaphoreType.DMA,
            pltpu.SemaphoreType.DMA,
        ],
    )
    def body(support_hbm, src_hbm, dst_hbm, ew_hbm, out_hbm,
             srcA, srcB, dstA, dstB, w_v, rowsA, rowsB, acc_sh, semA, semB):
        cid = lax.axis_index("c")
        sid = lax.axis_index("s")
        kc = jnp.where(cid == 0, K0, K1)
        nrounds = kc // 2
        base = (cid * NS * K0 + sid * kc) * CH
        pltpu.sync_copy(ew_hbm.at[pl.ds(base, wmax)], w_v)

        zvec = jnp.zeros((16,), jnp.float32)

        def zero_row(i, carry):
            for p in range(nseg):
                rowsA[i, pl.ds(p * 16, 16)] = zvec
            return carry

        lax.fori_loop(0, ZR, zero_row, 0)
        for k in range(rows_per_tile // ZR):
            pltpu.sync_copy(
                rowsA.at[pl.ds(0, ZR)],
                acc_sh.at[pl.ds(sid * rows_per_tile + k * ZR, ZR)])

        pltpu.sync_copy(src_hbm.at[pl.ds(base, CH)], srcA)
        pltpu.sync_copy(dst_hbm.at[pl.ds(base, CH)], dstA)
        plsc.subcore_barrier()
        pltpu.async_copy(support_hbm.at[srcA], rowsA, semA)

        def scale(rows_ref, ci):
            def scale_group(g, c2):
                wvec = w_v[pl.ds(ci * CH + g * 16, 16)]
                for l in range(16):
                    w = wvec[l]
                    j = g * 16 + l
                    for p in range(nseg):
                        sl = pl.ds(p * 16, 16)
                        rows_ref[j, sl] = rows_ref[j, sl] * w
                return c2

            lax.fori_loop(0, CH // 16, scale_group, 0)

        def round_body(r, carry):
            i0 = 2 * r
            i1 = i0 + 1
            pltpu.sync_copy(src_hbm.at[pl.ds(base + i1 * CH, CH)], srcB)
            pltpu.sync_copy(dst_hbm.at[pl.ds(base + i1 * CH, CH)], dstB)
            pltpu.make_async_copy(support_hbm.at[srcA], rowsA, semA).wait()
            pltpu.async_copy(support_hbm.at[srcB], rowsB, semB)
            scale(rowsA, i0)
            pltpu.sync_copy(rowsA, acc_sh.at[dstA], add=True)

            @pl.when(r < nrounds - 1)
            def _():
                pltpu.sync_copy(
                    src_hbm.at[pl.ds(base + (i0 + 2) * CH, CH)], srcA)
                pltpu.sync_copy(
                    dst_hbm.at[pl.ds(base + (i0 + 2) * CH, CH)], dstA)

            pltpu.make_async_copy(support_hbm.at[srcB], rowsB, semB).wait()

            @pl.when(r < nrounds - 1)
            def _():
                pltpu.async_copy(support_hbm.at[srcA], rowsA, semA)

            scale(rowsB, i1)
            pltpu.sync_copy(rowsB, acc_sh.at[dstB], add=True)
            return carry

        lax.fori_loop(0, nrounds, round_body, 0)
        plsc.subcore_barrier()

        main = (N // 8 // NS) * 8
        r0 = sid * main
        pltpu.sync_copy(acc_sh.at[pl.ds(r0, main)],
                        out_hbm.at[cid, pl.ds(r0, main)])
        rem = N - main * NS
        if rem:
            nrem = rem // 8

            @pl.when(sid < nrem)
            def _():
                rr = main * NS + sid * 8
                pltpu.sync_copy(acc_sh.at[pl.ds(rr, 8)],
                                out_hbm.at[cid, pl.ds(rr, 8)])

    return body(support, src, dst, ew)


def _tc_combine(partials, W):
    _, N, D = partials.shape
    DO = W.shape[1]
    BLK = 1000
    assert N % BLK == 0

    def body(p_ref, w_ref, o_ref):
        s = p_ref[0] + p_ref[1]
        o_ref[...] = jnp.dot(s, w_ref[...], preferred_element_type=jnp.float32)

    return pl.pallas_call(
        body,
        grid=(N // BLK,),
        in_specs=[
            pl.BlockSpec((2, BLK, D), lambda i: (0, i, 0)),
            pl.BlockSpec((D, DO), lambda i: (0, 0)),
        ],
        out_specs=pl.BlockSpec((BLK, DO), lambda i: (i, 0)),
        out_shape=jax.ShapeDtypeStruct((N, DO), jnp.float32),
    )(partials, W)


def kernel(support, edge_index, edge_weight, W):
    dst = edge_index[0].astype(jnp.int32)
    src = edge_index[1].astype(jnp.int32)
    partials = _sc_aggregate(support, src, dst, edge_weight)
    return _tc_combine(partials, W)


# speedup vs baseline: 1.3062x; 1.3062x over previous
import functools

import jax
import jax.numpy as jnp
from jax import lax
from jax.experimental import pallas as pl
from jax.experimental.pallas import tpu as pltpu
from jax.experimental.pallas import tpu_sc as plsc

NC = 2
NS = 16
NW = NC * NS
CH = 80
PACK = 14


def _sc_aggregate(support, src, dst, ew):
    N, D = support.shape
    E = src.shape[0]
    assert N <= (1 << PACK)
    unit = NW * CH * 2
    e_pad = -(-E // unit) * unit
    pad = e_pad - E
    if pad:
        src = jnp.pad(src, (0, pad))
        dst = jnp.pad(dst, (0, pad))
        ew = jnp.pad(ew, (0, pad))
    packed = src << PACK | dst
    epw = e_pad // NW
    nchunk = epw // CH
    nrounds = nchunk // 2
    assert N % NS == 0
    rows_per_tile = N // NS
    ZR = 125 if CH >= 125 else 25
    assert rows_per_tile % ZR == 0 and ZR <= CH
    nseg = D // 16

    mesh = plsc.VectorSubcoreMesh(core_axis_name="c", subcore_axis_name="s")

    @functools.partial(
        pl.kernel,
        mesh=mesh,
        out_type=jax.ShapeDtypeStruct((NC, N, D), jnp.float32),
        scratch_types=[
            pltpu.VMEM((epw,), jnp.int32),
            pltpu.VMEM((CH,), jnp.int32),
            pltpu.VMEM((CH,), jnp.int32),
            pltpu.VMEM((CH,), jnp.int32),
            pltpu.VMEM((CH,), jnp.int32),
            pltpu.VMEM((epw,), jnp.float32),
            pltpu.VMEM((CH, D), jnp.float32),
            pltpu.VMEM((CH, D), jnp.float32),
            pltpu.VMEM_SHARED((N, D), jnp.float32),
            pltpu.SemaphoreType.DMA,
            pltpu.SemaphoreType.DMA,
        ],
    )
    def body(support_hbm, packed_hbm, ew_hbm, out_hbm,
             pk_v, srcA, srcB, dstA, dstB, w_v, rowsA, rowsB, acc_sh,
             semA, semB):
        cid = lax.axis_index("c")
        sid = lax.axis_index("s")
        wid = sid * NC + cid
        base = wid * epw
        pltpu.sync_copy(ew_hbm.at[pl.ds(base, epw)], w_v)
        pltpu.sync_copy(packed_hbm.at[pl.ds(base, epw)], pk_v)

        zvec = jnp.zeros((16,), jnp.float32)

        def zero_row(i, carry):
            for p in range(nseg):
                rowsA[i, pl.ds(p * 16, 16)] = zvec
            return carry

        lax.fori_loop(0, ZR, zero_row, 0)
        for k in range(rows_per_tile // ZR):
            pltpu.sync_copy(
                rowsA.at[pl.ds(0, ZR)],
                acc_sh.at[pl.ds(sid * rows_per_tile + k * ZR, ZR)])

        def unpack(ci, src_ref, dst_ref):
            mask = jnp.full((16,), (1 << PACK) - 1, jnp.int32)

            def up(g, c2):
                pk = pk_v[pl.ds(ci * CH + g * 16, 16)]
                sl = pl.ds(g * 16, 16)
                src_ref[sl] = lax.shift_right_logical(pk, PACK)
                dst_ref[sl] = lax.bitwise_and(pk, mask)
                return c2

            lax.fori_loop(0, CH // 16, up, 0)

        unpack(0, srcA, dstA)
        plsc.subcore_barrier()
        pltpu.async_copy(support_hbm.at[srcA], rowsA, semA)

        def scale(rows_ref, ci):
            def scale_group(g, c2):
                wvec = w_v[pl.ds(ci * CH + g * 16, 16)]
                for l in range(16):
                    w = wvec[l]
                    j = g * 16 + l
                    for p in range(nseg):
                        sl = pl.ds(p * 16, 16)
                        rows_ref[j, sl] = rows_ref[j, sl] * w
                return c2

            lax.fori_loop(0, CH // 16, scale_group, 0)

        def round_body(r, carry):
            i0 = 2 * r
            i1 = i0 + 1
            unpack(i1, srcB, dstB)
            pltpu.make_async_copy(support_hbm.at[srcA], rowsA, semA).wait()
            pltpu.async_copy(support_hbm.at[srcB], rowsB, semB)
            scale(rowsA, i0)
            pltpu.sync_copy(rowsA, acc_sh.at[dstA], add=True)

            @pl.when(r < nrounds - 1)
            def _():
                unpack(i0 + 2, srcA, dstA)

            pltpu.make_async_copy(support_hbm.at[srcB], rowsB, semB).wait()

            @pl.when(r < nrounds - 1)
            def _():
                pltpu.async_copy(support_hbm.at[srcA], rowsA, semA)

            scale(rowsB, i1)
            pltpu.sync_copy(rowsB, acc_sh.at[dstB], add=True)
            return carry

        lax.fori_loop(0, nrounds, round_body, 0)
        plsc.subcore_barrier()

        main = (N // 8 // NS) * 8
        r0 = sid * main
        pltpu.sync_copy(acc_sh.at[pl.ds(r0, main)],
                        out_hbm.at[cid, pl.ds(r0, main)])
        rem = N - main * NS
        if rem:
            nrem = rem // 8

            @pl.when(sid < nrem)
            def _():
                rr = main * NS + sid * 8
                pltpu.sync_copy(acc_sh.at[pl.ds(rr, 8)],
                                out_hbm.at[cid, pl.ds(rr, 8)])

    return body(support, packed, ew)


def _tc_combine(partials, W):
    _, N, D = partials.shape
    DO = W.shape[1]
    BLK = 1000
    assert N % BLK == 0

    def body(p_ref, w_ref, o_ref):
        s = p_ref[0] + p_ref[1]
        o_ref[...] = jnp.dot(s, w_ref[...], preferred_element_type=jnp.float32)

    return pl.pallas_call(
        body,
        grid=(N // BLK,),
        in_specs=[
            pl.BlockSpec((2, BLK, D), lambda i: (0, i, 0)),
            pl.BlockSpec((D, DO), lambda i: (0, 0)),
        ],
        out_specs=pl.BlockSpec((BLK, DO), lambda i: (i, 0)),
        out_shape=jax.ShapeDtypeStruct((N, DO), jnp.float32),
    )(partials, W)


def kernel(support, edge_index, edge_weight, W):
    dst = edge_index[0].astype(jnp.int32)
    src = edge_index[1].astype(jnp.int32)
    partials = _sc_aggregate(support, src, dst, edge_weight)
    return _tc_combine(partials, W)
